# Initial kernel scaffold; baseline (speedup 1.0000x reference)
import jax
import jax.numpy as jnp
from jax.experimental import pallas as pl


def kernel(x, edge_index, batch, Wl1, Wr1, b1, Wl2, Wr2, b2, Wl3, Wr3, b3, W_lin, b_lin):
    raise NotImplementedError("write your pallas kernel here")


# speedup vs baseline: 10.7385x; 10.7385x over previous
import functools

import jax
import jax.numpy as jnp
from jax import lax
from jax.experimental import pallas as pl
from jax.experimental.pallas import tpu as pltpu
from jax.experimental.pallas import tpu_sc as plsc

_N = 10000
_E = 320000
_D = 128
_B = 64
_C = 10

_NCORES = 2
_NSUB = 16
_NW = _NCORES * _NSUB

_NP = 10240
_CHUNK = 128
_CPT = 80
_IB = 40
_NIB = _CPT // _IB
_EP = _NW * _CPT * _CHUNK
_RPT = _NP // _NSUB

_BN = 256
_NBLK = _NP // _BN


def _sc_agg(with_deg):
    mesh = plsc.VectorSubcoreMesh(
        core_axis_name="c", subcore_axis_name="s",
        num_cores=_NCORES, num_subcores=_NSUB)

    def body(u_hbm, src_hbm, dst_hbm, *rest):
        if with_deg:
            (out_hbm, deg_hbm,
             srcv, dstv, rows, onesv, zb, agg, sdeg, sem0, sem1) = rest
        else:
            (out_hbm, srcv, dstv, rows, agg, sem0, sem1) = rest
        cid = lax.axis_index("c")
        sid = lax.axis_index("s")
        wid = cid * _NSUB + sid

        zero = jnp.zeros((16,), jnp.float32)

        def zrow(r, carry):
            for c in range(_D // 16):
                rows[0, r, c * 16:(c + 1) * 16] = zero
            return carry

        lax.fori_loop(0, _CHUNK, zrow, 0)
        base = sid * _RPT
        for k in range(_RPT // _CHUNK):
            pltpu.sync_copy(rows.at[0], agg.at[pl.ds(base + k * _CHUNK, _CHUNK)])
        if with_deg:
            def zdeg(i, carry):
                onesv[pl.ds(i * 16, 16)] = zero + 1.0
                return carry

            lax.fori_loop(0, _CHUNK // 16, zdeg, 0)

            def zdeg2(i, carry):
                zb[pl.ds(i * 16, 16)] = zero
                return carry

            lax.fori_loop(0, _RPT // 16, zdeg2, 0)
            pltpu.sync_copy(zb, sdeg.at[pl.ds(base, _RPT)])
        plsc.subcore_barrier()

        for ib in range(_NIB):
            pltpu.sync_copy(src_hbm.at[pl.ds(wid * _CPT + ib * _IB, _IB)],
                            srcv)
            pltpu.sync_copy(dst_hbm.at[pl.ds(wid * _CPT + ib * _IB, _IB)],
                            dstv)
            pltpu.async_copy(u_hbm.at[srcv.at[0]], rows.at[0], sem0)
            pltpu.async_copy(u_hbm.at[srcv.at[1]], rows.at[1], sem1)

            def step(j2, carry):
                a = 2 * j2
                pltpu.make_async_copy(
                    u_hbm.at[srcv.at[0]], rows.at[0], sem0).wait()
                pltpu.sync_copy(rows.at[0], agg.at[dstv.at[a]], add=True)
                if with_deg:
                    pltpu.sync_copy(onesv, sdeg.at[dstv.at[a]], add=True)
                na = jnp.minimum(a + 2, _IB - 1)
                pltpu.async_copy(u_hbm.at[srcv.at[na]], rows.at[0], sem0)
                b = a + 1
                pltpu.make_async_copy(
                    u_hbm.at[srcv.at[1]], rows.at[1], sem1).wait()
                pltpu.sync_copy(rows.at[1], agg.at[dstv.at[b]], add=True)
                if with_deg:
                    pltpu.sync_copy(onesv, sdeg.at[dstv.at[b]], add=True)
                nb = jnp.minimum(b + 2, _IB - 1)
                pltpu.async_copy(u_hbm.at[srcv.at[nb]], rows.at[1], sem1)
                return carry

            lax.fori_loop(0, _IB // 2, step, 0)
            pltpu.make_async_copy(u_hbm.at[srcv.at[0]], rows.at[0], sem0).wait()
            pltpu.make_async_copy(u_hbm.at[srcv.at[1]], rows.at[1], sem1).wait()
        plsc.subcore_barrier()

        pltpu.sync_copy(agg.at[pl.ds(base, _RPT)],
                        out_hbm.at[cid, pl.ds(base, _RPT)])
        if with_deg:
            pltpu.sync_copy(sdeg.at[pl.ds(base, _RPT)],
                            deg_hbm.at[cid, pl.ds(base, _RPT)])

    out_type = [jax.ShapeDtypeStruct((_NCORES, _NP, _D), jnp.float32)]
    scratch = [
        pltpu.VMEM((_IB, _CHUNK), jnp.int32),
        pltpu.VMEM((_IB, _CHUNK), jnp.int32),
        pltpu.VMEM((2, _CHUNK, _D), jnp.float32),
    ]
    if with_deg:
        out_type.append(jax.ShapeDtypeStruct((_NCORES, _NP), jnp.float32))
        scratch.append(pltpu.VMEM((_CHUNK,), jnp.float32))
        scratch.append(pltpu.VMEM((_RPT,), jnp.float32))
    scratch.append(pltpu.VMEM_SHARED((_NP, _D), jnp.float32))
    if with_deg:
        scratch.append(pltpu.VMEM_SHARED((_NP,), jnp.float32))
    scratch += [pltpu.SemaphoreType.DMA, pltpu.SemaphoreType.DMA]

    return pl.kernel(
        body,
        out_type=tuple(out_type) if with_deg else out_type[0],
        mesh=mesh,
        scratch_types=scratch,
    )


def _t0_body(x_ref, wl_ref, o_ref):
    o_ref[...] = jnp.dot(x_ref[...], wl_ref[...],
                         preferred_element_type=jnp.float32)


def _t1_body(s_ref, deg_ref, x_ref, wr_ref, b_ref, wl2_ref,
             h_ref, u_ref, dinv_ref):
    agg = s_ref[0] + s_ref[1]
    deg = deg_ref[0, 0] + deg_ref[1, 0]
    dinv = 1.0 / jnp.maximum(deg, 1.0)
    xr = jnp.dot(x_ref[...], wr_ref[...], preferred_element_type=jnp.float32)
    h = jnp.maximum(agg * dinv + xr + b_ref[...], 0.0)
    h_ref[...] = h
    u_ref[...] = jnp.dot(h, wl2_ref[...], preferred_element_type=jnp.float32)
    dinv_ref[...] = jnp.broadcast_to(dinv, (_BN, 16))


def _t2_body(s_ref, h_ref, dinv_ref, wr_ref, b_ref, wl3_ref, ho_ref, u_ref):
    s = s_ref[0] + s_ref[1]
    dinv = jnp.max(dinv_ref[...], axis=1, keepdims=True)
    hr = jnp.dot(h_ref[...], wr_ref[...], preferred_element_type=jnp.float32)
    h = jnp.maximum(s * dinv + hr + b_ref[...], 0.0)
    ho_ref[...] = h
    u_ref[...] = jnp.dot(h, wl3_ref[...], preferred_element_type=jnp.float32)


def _t3_body(s_ref, h_ref, dinv_ref, wr_ref, b_ref, brow_ref, bcol_ref,
             wa_ref, wb_ref, blin_ref, o_ref, gmax, gsum, gcnt):
    i = pl.program_id(0)
    nb = pl.num_programs(0)

    @pl.when(i == 0)
    def _init():
        gmax[...] = jnp.full((_B, 128), -jnp.inf, jnp.float32)
        gsum[...] = jnp.zeros((_B, 128), jnp.float32)
        gcnt[...] = jnp.zeros((_B, 128), jnp.float32)

    s = s_ref[0] + s_ref[1]
    dinv = jnp.max(dinv_ref[...], axis=1, keepdims=True)
    hr = jnp.dot(h_ref[...], wr_ref[...], preferred_element_type=jnp.float32)
    h3 = s * dinv + hr + b_ref[...]

    brow = brow_ref[0]
    onehot = (lax.broadcasted_iota(jnp.int32, (_B, _BN), 0)
              == brow).astype(jnp.float32)
    gsum[...] += jnp.dot(onehot, h3, preferred_element_type=jnp.float32)
    gcnt[...] += jnp.broadcast_to(
        jnp.sum(onehot, axis=1, keepdims=True), (_B, 128))

    bcol = bcol_ref[0]
    lo = jnp.min(bcol)
    hi = jnp.max(bcol)
    giota = lax.broadcasted_iota(jnp.int32, (_B, 1), 0)

    def upd(b, carry):
        m = bcol == b
        contrib = jnp.max(jnp.where(m, h3, -jnp.inf), axis=0, keepdims=True)
        sel = giota == b
        cur = gmax[...]
        gmax[...] = jnp.where(sel, jnp.maximum(cur, contrib), cur)
        return carry

    lax.fori_loop(lo, hi + 1, upd, 0)

    @pl.when(i == nb - 1)
    def _fin():
        mean = gsum[...] / jnp.maximum(gcnt[...], 1.0)
        o_ref[...] = (
            jnp.dot(gmax[...], wa_ref[...], preferred_element_type=jnp.float32)
            + jnp.dot(mean, wb_ref[...], preferred_element_type=jnp.float32)
            + blin_ref[...])


def _node_spec(w):
    return pl.BlockSpec((_BN, w), lambda i: (i, 0))


def _full_spec(shape):
    nd = len(shape)
    return pl.BlockSpec(shape, lambda i, _nd=nd: (0,) * _nd)


def _s_spec(w):
    return pl.BlockSpec((_NCORES, _BN, w), lambda i: (0, i, 0))


def kernel(x, edge_index, batch, Wl1, Wr1, b1, Wl2, Wr2, b2, Wl3, Wr3, b3,
           W_lin, b_lin):
    x_p = jnp.zeros((_NP, _D), jnp.float32).at[:_N].set(x)
    pad = _EP - _E
    pi = jnp.arange(pad, dtype=jnp.int32)
    src_rows = jnp.concatenate(
        [edge_index[0], pi % _N]).reshape(_NW * _CPT, _CHUNK)
    dst_rows = jnp.concatenate(
        [edge_index[1], _N + pi % (_NP - _N)]).reshape(_NW * _CPT, _CHUNK)
    batch_p = jnp.concatenate(
        [batch.astype(jnp.int32), jnp.full((_NP - _N,), _B, jnp.int32)])
    brow = batch_p.reshape(_NBLK, 1, _BN)
    bcol = batch_p.reshape(_NBLK, _BN, 1)
    b1r = b1.reshape(1, _D)
    b2r = b2.reshape(1, _D)
    b3r = b3.reshape(1, _D)
    blinr = b_lin.reshape(1, _C)
    wa = W_lin[:_D]
    wb = W_lin[_D:]

    u1 = pl.pallas_call(
        _t0_body,
        grid=(_NBLK,),
        in_specs=[_node_spec(_D), _full_spec((_D, _D))],
        out_specs=_node_spec(_D),
        out_shape=jax.ShapeDtypeStruct((_NP, _D), jnp.float32),
    )(x_p, Wl1)

    s1, deg = _sc_agg(True)(u1, src_rows, dst_rows)
    deg4 = deg.reshape(_NCORES, _NBLK, _BN, 1)

    h1, u2, dinv = pl.pallas_call(
        _t1_body,
        grid=(_NBLK,),
        in_specs=[_s_spec(_D),
                  pl.BlockSpec((_NCORES, 1, _BN, 1), lambda i: (0, i, 0, 0)),
                  _node_spec(_D), _full_spec((_D, _D)),
                  _full_spec((1, _D)), _full_spec((_D, _D))],
        out_specs=[_node_spec(_D), _node_spec(_D), _node_spec(16)],
        out_shape=[jax.ShapeDtypeStruct((_NP, _D), jnp.float32),
                   jax.ShapeDtypeStruct((_NP, _D), jnp.float32),
                   jax.ShapeDtypeStruct((_NP, 16), jnp.float32)],
    )(s1, deg4, x_p, Wr1, b1r, Wl2)

    s2 = _sc_agg(False)(u2, src_rows, dst_rows)

    h2, u3 = pl.pallas_call(
        _t2_body,
        grid=(_NBLK,),
        in_specs=[_s_spec(_D), _node_spec(_D), _node_spec(16),
                  _full_spec((_D, _D)), _full_spec((1, _D)),
                  _full_spec((_D, _D))],
        out_specs=[_node_spec(_D), _node_spec(_D)],
        out_shape=[jax.ShapeDtypeStruct((_NP, _D), jnp.float32),
                   jax.ShapeDtypeStruct((_NP, _D), jnp.float32)],
    )(s2, h1, dinv, Wr2, b2r, Wl3)

    s3 = _sc_agg(False)(u3, src_rows, dst_rows)

    out = pl.pallas_call(
        _t3_body,
        grid=(_NBLK,),
        in_specs=[_s_spec(_D), _node_spec(_D), _node_spec(16),
                  _full_spec((_D, _D)), _full_spec((1, _D)),
                  pl.BlockSpec((1, 1, _BN), lambda i: (i, 0, 0)),
                  pl.BlockSpec((1, _BN, 1), lambda i: (i, 0, 0)),
                  _full_spec((_D, _C)), _full_spec((_D, _C)),
                  _full_spec((1, _C))],
        out_specs=pl.BlockSpec((_B, _C), lambda i: (0, 0)),
        out_shape=jax.ShapeDtypeStruct((_B, _C), jnp.float32),
        scratch_shapes=[pltpu.VMEM((_B, 128), jnp.float32),
                        pltpu.VMEM((_B, 128), jnp.float32),
                        pltpu.VMEM((_B, 128), jnp.float32)],
    )(s3, h2, dinv, Wr3, b3r, brow, bcol, wa, wb, blinr)
    return out
